# Initial kernel scaffold; baseline (speedup 1.0000x reference)
import jax
import jax.numpy as jnp
from jax.experimental import pallas as pl


def kernel(loc_data, conf_data, priors, targets):
    raise NotImplementedError("write your pallas kernel here")


# speedup vs baseline: 14.7214x; 14.7214x over previous
import jax
import jax.numpy as jnp
from jax.experimental import pallas as pl
from jax.experimental.pallas import tpu as pltpu

_NUM_CLASSES = 21
_B, _P, _O = 32, 8732, 8
_ROWS, _LANES = 70, 128
_PP = _ROWS * _LANES
_THRESHOLD = 0.5
_V0, _V1 = 0.1, 0.2


def _body(targets_ref, loc_ref, conf_ref, pri_ref, out_l_ref, out_c_ref, acc_ref):
    i = pl.program_id(0)

    @pl.when(i == 0)
    def _():
        acc_ref[0] = 0.0
        acc_ref[1] = 0.0
        acc_ref[2] = 0.0

    lin = (jax.lax.broadcasted_iota(jnp.int32, (_ROWS, _LANES), 0) * _LANES
           + jax.lax.broadcasted_iota(jnp.int32, (_ROWS, _LANES), 1)
           ).astype(jnp.float32)
    valid = lin < float(_P)

    pcx = pri_ref[0]
    pcy = pri_ref[1]
    pw = pri_ref[2]
    ph = pri_ref[3]
    px1 = pcx - pw * 0.5
    py1 = pcy - ph * 0.5
    px2 = pcx + pw * 0.5
    py2 = pcy + ph * 0.5
    area_p = (px2 - px1) * (py2 - py1)

    bt_ov = jnp.full((_ROWS, _LANES), -1.0, jnp.float32)
    bt_idx = jnp.zeros((_ROWS, _LANES), jnp.float32)
    bpi = []
    tco = []
    for j in range(_O):
        tx1 = targets_ref[0, j, 0]
        ty1 = targets_ref[0, j, 1]
        tx2 = targets_ref[0, j, 2]
        ty2 = targets_ref[0, j, 3]
        lab = targets_ref[0, j, 4]
        tco.append((tx1, ty1, tx2, ty2, lab))
        iw = jnp.maximum(jnp.minimum(px2, tx2) - jnp.maximum(px1, tx1), 0.0)
        ih = jnp.maximum(jnp.minimum(py2, ty2) - jnp.maximum(py1, ty1), 0.0)
        inter = iw * ih
        area_t = (tx2 - tx1) * (ty2 - ty1)
        ov = inter / (area_t + area_p - inter)
        ov = jnp.where(valid, ov, -1.0)
        upd = ov > bt_ov
        bt_ov = jnp.where(upd, ov, bt_ov)
        bt_idx = jnp.where(upd, float(j), bt_idx)
        m = jnp.max(ov)
        bpi.append(jnp.min(jnp.where(ov == m, lin, float(_PP))))

    for j in range(_O):
        f = lin == bpi[j]
        bt_ov = jnp.where(f, 2.0, bt_ov)
        bt_idx = jnp.where(f, float(j), bt_idx)

    mx1 = jnp.zeros_like(lin)
    my1 = jnp.zeros_like(lin)
    mx2 = jnp.zeros_like(lin)
    my2 = jnp.zeros_like(lin)
    mlab = jnp.zeros_like(lin)
    for j in range(_O):
        sel = bt_idx == float(j)
        tx1, ty1, tx2, ty2, lab = tco[j]
        mx1 = jnp.where(sel, tx1, mx1)
        my1 = jnp.where(sel, ty1, my1)
        mx2 = jnp.where(sel, tx2, mx2)
        my2 = jnp.where(sel, ty2, my2)
        mlab = jnp.where(sel, lab, mlab)

    cf = jnp.where(bt_ov < _THRESHOLD, 0.0, mlab + 1.0)
    pos = cf > 0.0
    posf = pos.astype(jnp.float32)

    g_cx = ((mx1 + mx2) * 0.5 - pcx) / (_V0 * pw)
    g_cy = ((my1 + my2) * 0.5 - pcy) / (_V0 * ph)
    g_w = jnp.log((mx2 - mx1) / pw) / _V1
    g_h = jnp.log((my2 - my1) / ph) / _V1
    sl = jnp.zeros_like(lin)
    for c, g in enumerate((g_cx, g_cy, g_w, g_h)):
        d = jnp.abs(loc_ref[0, c] - g)
        sl = sl + jnp.where(d < 1.0, 0.5 * d * d, d - 0.5)
    loss_l_img = jnp.sum(sl * posf)

    cmax = conf_ref[0, 0]
    for c in range(1, _NUM_CLASSES):
        cmax = jnp.maximum(cmax, conf_ref[0, c])
    s = jnp.zeros_like(lin)
    g = jnp.zeros_like(lin)
    for c in range(_NUM_CLASSES):
        row = conf_ref[0, c]
        s = s + jnp.exp(row - cmax)
        g = jnp.where(cf == float(c), row, g)
    lse = jnp.log(s) + cmax
    lca = jnp.where(valid, lse - g, 0.0)
    lcr = jnp.where(pos, 0.0, lca)

    num_pos = jnp.sum(posf)
    sum_pos_lca = jnp.sum(lca * posf)
    k = jnp.minimum(3.0 * num_pos, float(_P - 1))

    bits = jax.lax.bitcast_convert_type(lcr, jnp.int32)

    def bs_body(it, t):
        cand = t | (jnp.int32(1) << (jnp.int32(30) - it))
        cnt = jnp.sum(jnp.where(bits >= cand, 1.0, 0.0))
        return jnp.where(cnt >= k, cand, t)

    t = jax.lax.fori_loop(0, 31, bs_body, jnp.int32(0))
    tval = jax.lax.bitcast_convert_type(t, jnp.float32)
    gt = lcr > tval
    m_cnt = jnp.sum(gt.astype(jnp.float32))
    sum_gt = jnp.sum(jnp.where(gt, lcr, 0.0))
    loss_c_img = sum_pos_lca + sum_gt + (k - m_cnt) * tval

    acc_ref[0] = acc_ref[0] + loss_l_img
    acc_ref[1] = acc_ref[1] + loss_c_img
    acc_ref[2] = acc_ref[2] + num_pos

    @pl.when(i == _B - 1)
    def _():
        n = acc_ref[2]
        out_l_ref[0, 0] = acc_ref[0] / n
        out_c_ref[0, 0] = acc_ref[1] / n


def kernel(loc_data, conf_data, priors, targets):
    pad = _PP - _P
    loc_p = jnp.pad(jnp.swapaxes(loc_data, 1, 2),
                    ((0, 0), (0, 0), (0, pad))).reshape(_B, 4, _ROWS, _LANES)
    conf_p = jnp.pad(jnp.swapaxes(conf_data, 1, 2),
                     ((0, 0), (0, 0), (0, pad))).reshape(_B, _NUM_CLASSES, _ROWS, _LANES)
    pri_p = jnp.pad(priors.T, ((0, 0), (0, pad)),
                    constant_values=0.5).reshape(4, _ROWS, _LANES)

    out_l, out_c = pl.pallas_call(
        _body,
        grid=(_B,),
        in_specs=[
            pl.BlockSpec((1, _O, 5), lambda i: (i, 0, 0), memory_space=pltpu.SMEM),
            pl.BlockSpec((1, 4, _ROWS, _LANES), lambda i: (i, 0, 0, 0)),
            pl.BlockSpec((1, _NUM_CLASSES, _ROWS, _LANES), lambda i: (i, 0, 0, 0)),
            pl.BlockSpec((4, _ROWS, _LANES), lambda i: (0, 0, 0)),
        ],
        out_specs=[
            pl.BlockSpec((1, 1), lambda i: (0, 0), memory_space=pltpu.SMEM),
            pl.BlockSpec((1, 1), lambda i: (0, 0), memory_space=pltpu.SMEM),
        ],
        out_shape=[jax.ShapeDtypeStruct((1, 1), jnp.float32)] * 2,
        scratch_shapes=[pltpu.SMEM((3,), jnp.float32)],
    )(targets, loc_p, conf_p, pri_p)
    return out_l[0, 0], out_c[0, 0]


# speedup vs baseline: 32.3327x; 2.1963x over previous
import jax
import jax.numpy as jnp
from jax.experimental import pallas as pl
from jax.experimental.pallas import tpu as pltpu

_NUM_CLASSES = 21
_B, _P, _O = 32, 8732, 8
_ROWS, _LANES = 70, 128
_PP = _ROWS * _LANES
_THRESHOLD = 0.5
_V0, _V1 = 0.1, 0.2
_IMGS = 4
_STEPS = _B // _IMGS


def _body(targets_ref, loc_ref, conf_ref, pri_ref, out_ref, acc_ref, ov_scr,
          lcr_scr, meta_scr):
    i = pl.program_id(0)

    @pl.when(i == 0)
    def _():
        acc_ref[...] = jnp.zeros((8, _LANES), jnp.float32)

    cur = jax.lax.rem(i, 2)
    prv = jax.lax.rem(i + 1, 2)

    ones = jnp.ones((_LANES, _LANES), jnp.float32)
    ones_bf = jnp.ones((_LANES, _LANES), jnp.bfloat16)

    def rowsums(rows):
        stacked = jnp.concatenate(rows, axis=0)
        return jax.lax.dot_general(stacked, ones, (((1,), (0,)), ((), ())),
                                   preferred_element_type=jnp.float32)

    def rowcounts(rows):
        stacked = jnp.concatenate(rows, axis=0).astype(jnp.bfloat16)
        return jax.lax.dot_general(stacked, ones_bf, (((1,), (0,)), ((), ())),
                                   preferred_element_type=jnp.float32)

    def part(x):
        return jnp.sum(x, axis=0, keepdims=True)

    meta = meta_scr[prv]
    ks = [jax.lax.slice(meta, (m, 0), (m + 1, _LANES)) for m in range(_IMGS)]
    splca = [jax.lax.slice(meta, (_IMGS + m, 0), (_IMGS + m + 1, _LANES))
             for m in range(_IMGS)]
    bits = [jax.lax.bitcast_convert_type(lcr_scr[prv, m], jnp.int32)
            for m in range(_IMGS)]
    ts = [jnp.zeros((1, _LANES), jnp.int32) for _ in range(_IMGS)]

    def partcount(m, cnd):
        mask = bits[m] >= jnp.broadcast_to(cnd, (_ROWS, _LANES))
        return part(mask.astype(jnp.float32))

    def do_level(lvl):
        if lvl == 0:
            c30 = [t + (1 << 30) for t in ts]
            cnts = rowcounts([partcount(m, c30[m]) for m in range(_IMGS)])
            for m in range(_IMGS):
                ok = jax.lax.slice(cnts, (m, 0), (m + 1, _LANES)) >= ks[m]
                ts[m] = jnp.where(ok, c30[m], ts[m])
            return
        lo = 28 - 2 * (lvl - 1)
        cands = [[t + (d << lo) for d in (1, 2, 3)] for t in ts]
        parts = [partcount(m, cands[m][d]) for m in range(_IMGS)
                 for d in range(3)]
        cnts = rowcounts(parts)
        for m in range(_IMGS):
            ok1 = jax.lax.slice(cnts, (3 * m, 0), (3 * m + 1, _LANES)) >= ks[m]
            ok2 = jax.lax.slice(cnts, (3 * m + 1, 0), (3 * m + 2, _LANES)) >= ks[m]
            ok3 = jax.lax.slice(cnts, (3 * m + 2, 0), (3 * m + 3, _LANES)) >= ks[m]
            ts[m] = jnp.where(ok3, cands[m][2],
                              jnp.where(ok2, cands[m][1],
                                        jnp.where(ok1, cands[m][0], ts[m])))

    lvl_counter = [0]

    def step_level():
        if lvl_counter[0] < 16:
            do_level(lvl_counter[0])
            lvl_counter[0] += 1

    lin = (jax.lax.broadcasted_iota(jnp.int32, (_ROWS, _LANES), 0) * _LANES
           + jax.lax.broadcasted_iota(jnp.int32, (_ROWS, _LANES), 1)
           ).astype(jnp.float32)
    valid = lin < float(_P)

    pcx = pri_ref[0]
    pcy = pri_ref[1]
    pw = pri_ref[2]
    ph = pri_ref[3]
    px1 = pcx - pw * 0.5
    py1 = pcy - ph * 0.5
    px2 = pcx + pw * 0.5
    py2 = pcy + ph * 0.5
    area_p = (px2 - px1) * (py2 - py1)

    ra_parts = []
    for m in range(_IMGS):
        bt_ov = jnp.full((_ROWS, _LANES), -1.0, jnp.float32)
        bt_idx = jnp.zeros((_ROWS, _LANES), jnp.float32)
        tco = []
        pmax = []
        for j in range(_O):
            tx1 = targets_ref[m, j, 0]
            ty1 = targets_ref[m, j, 1]
            tx2 = targets_ref[m, j, 2]
            ty2 = targets_ref[m, j, 3]
            lab = targets_ref[m, j, 4]
            tco.append((tx1, ty1, tx2, ty2, lab))
            iw = jnp.maximum(jnp.minimum(px2, tx2) - jnp.maximum(px1, tx1), 0.0)
            ih = jnp.maximum(jnp.minimum(py2, ty2) - jnp.maximum(py1, ty1), 0.0)
            inter = iw * ih
            area_t = (tx2 - tx1) * (ty2 - ty1)
            ov = inter / (area_t + area_p - inter)
            ov = jnp.where(valid, ov, -1.0)
            ov_scr[j] = ov
            upd = ov > bt_ov
            bt_ov = jnp.where(upd, ov, bt_ov)
            bt_idx = jnp.where(upd, float(j), bt_idx)
            pmax.append(jnp.max(ov, axis=0, keepdims=True))

        step_level()

        m8 = jnp.max(jnp.concatenate(pmax, axis=0), axis=1, keepdims=True)
        m8b = jnp.broadcast_to(m8, (_O, _LANES))
        pidx = []
        for j in range(_O):
            mj = jnp.broadcast_to(
                jax.lax.slice(m8b, (j, 0), (j + 1, _LANES)), (_ROWS, _LANES))
            cand = jnp.where(ov_scr[j] == mj, lin, float(_PP))
            pidx.append(jnp.min(cand, axis=0, keepdims=True))
        bpi8 = jnp.min(jnp.concatenate(pidx, axis=0), axis=1, keepdims=True)
        bpi8b = jnp.broadcast_to(bpi8, (_O, _LANES))

        for j in range(_O):
            bj = jnp.broadcast_to(
                jax.lax.slice(bpi8b, (j, 0), (j + 1, _LANES)), (_ROWS, _LANES))
            f = lin == bj
            bt_ov = jnp.where(f, 2.0, bt_ov)
            bt_idx = jnp.where(f, float(j), bt_idx)

        step_level()

        mx1 = jnp.zeros_like(lin)
        my1 = jnp.zeros_like(lin)
        mx2 = jnp.zeros_like(lin)
        my2 = jnp.zeros_like(lin)
        mlab = jnp.zeros_like(lin)
        for j in range(_O):
            sel = bt_idx == float(j)
            tx1, ty1, tx2, ty2, lab = tco[j]
            mx1 = jnp.where(sel, tx1, mx1)
            my1 = jnp.where(sel, ty1, my1)
            mx2 = jnp.where(sel, tx2, mx2)
            my2 = jnp.where(sel, ty2, my2)
            mlab = jnp.where(sel, lab, mlab)

        cf = jnp.where(bt_ov < _THRESHOLD, 0.0, mlab + 1.0)
        pos = cf > 0.0
        posf = pos.astype(jnp.float32)

        g_cx = ((mx1 + mx2) * 0.5 - pcx) / (_V0 * pw)
        g_cy = ((my1 + my2) * 0.5 - pcy) / (_V0 * ph)
        g_w = jnp.log((mx2 - mx1) / pw) / _V1
        g_h = jnp.log((my2 - my1) / ph) / _V1
        sl = jnp.zeros_like(lin)
        for c, g in enumerate((g_cx, g_cy, g_w, g_h)):
            d = jnp.abs(loc_ref[m, c] - g)
            sl = sl + jnp.where(d < 1.0, 0.5 * d * d, d - 0.5)

        step_level()

        cmax = conf_ref[m, 0]
        for c in range(1, _NUM_CLASSES):
            cmax = jnp.maximum(cmax, conf_ref[m, c])
        s = jnp.zeros_like(lin)
        g = jnp.zeros_like(lin)
        for c in range(_NUM_CLASSES):
            row = conf_ref[m, c]
            s = s + jnp.exp(row - cmax)
            g = jnp.where(cf == float(c), row, g)
        lse = jnp.log(s) + cmax
        lca = jnp.where(valid, lse - g, 0.0)
        lcr_scr[cur, m] = jnp.where(pos, 0.0, lca)

        ra_parts += [part(posf), part(lca * posf), part(sl * posf)]

        step_level()

    while lvl_counter[0] < 16:
        step_level()

    rb_parts = []
    tvals = []
    for m in range(_IMGS):
        tval = jax.lax.bitcast_convert_type(ts[m], jnp.float32)
        tvals.append(tval)
        lcr = lcr_scr[prv, m]
        gt = lcr > jnp.broadcast_to(tval, (_ROWS, _LANES))
        gtf = gt.astype(jnp.float32)
        rb_parts += [part(gtf), part(gtf * lcr)]
    rb = rowsums(rb_parts)

    lossc_t = jnp.zeros((1, _LANES), jnp.float32)
    for m in range(_IMGS):
        m_cnt = jax.lax.slice(rb, (2 * m, 0), (2 * m + 1, _LANES))
        sum_gt = jax.lax.slice(rb, (2 * m + 1, 0), (2 * m + 2, _LANES))
        lossc_t = lossc_t + splca[m] + sum_gt + (ks[m] - m_cnt) * tvals[m]
    lossc_t = jnp.where(i > 0, lossc_t, jnp.zeros_like(lossc_t))

    ra = rowsums(ra_parts)
    lossl_t = jnp.zeros((1, _LANES), jnp.float32)
    npos_t = jnp.zeros((1, _LANES), jnp.float32)
    meta_rows = []
    for m in range(_IMGS):
        num_pos = jax.lax.slice(ra, (3 * m, 0), (3 * m + 1, _LANES))
        npos_t = npos_t + num_pos
        lossl_t = lossl_t + jax.lax.slice(ra, (3 * m + 2, 0), (3 * m + 3, _LANES))
        meta_rows.append(jnp.minimum(3.0 * num_pos, float(_P - 1)))
    for m in range(_IMGS):
        meta_rows.append(jax.lax.slice(ra, (3 * m + 1, 0), (3 * m + 2, _LANES)))
    meta_scr[cur] = jnp.concatenate(meta_rows, axis=0)

    zrow = jnp.zeros((1, _LANES), jnp.float32)
    contrib = jnp.concatenate(
        [jnp.where(i < _STEPS, lossl_t, zrow), lossc_t,
         jnp.where(i < _STEPS, npos_t, zrow),
         jnp.zeros((5, _LANES), jnp.float32)], axis=0)
    acc = acc_ref[...] + contrib
    acc_ref[...] = acc

    @pl.when(i == _STEPS)
    def _():
        n = jnp.broadcast_to(jax.lax.slice(acc, (2, 0), (3, _LANES)),
                             (8, _LANES))
        out_ref[...] = acc / n


def kernel(loc_data, conf_data, priors, targets):
    pad = _PP - _P
    loc_p = jnp.pad(jnp.swapaxes(loc_data, 1, 2),
                    ((0, 0), (0, 0), (0, pad))).reshape(_B, 4, _ROWS, _LANES)
    conf_p = jnp.pad(jnp.swapaxes(conf_data, 1, 2),
                     ((0, 0), (0, 0), (0, pad))).reshape(_B, _NUM_CLASSES, _ROWS, _LANES)
    pri_p = jnp.pad(priors.T, ((0, 0), (0, pad)),
                    constant_values=0.5).reshape(4, _ROWS, _LANES)

    def clamp(i):
        return jnp.minimum(i, _STEPS - 1)

    out = pl.pallas_call(
        _body,
        grid=(_STEPS + 1,),
        in_specs=[
            pl.BlockSpec((_IMGS, _O, 5), lambda i: (clamp(i), 0, 0),
                         memory_space=pltpu.SMEM),
            pl.BlockSpec((_IMGS, 4, _ROWS, _LANES),
                         lambda i: (clamp(i), 0, 0, 0)),
            pl.BlockSpec((_IMGS, _NUM_CLASSES, _ROWS, _LANES),
                         lambda i: (clamp(i), 0, 0, 0)),
            pl.BlockSpec((4, _ROWS, _LANES), lambda i: (0, 0, 0)),
        ],
        out_specs=pl.BlockSpec((8, _LANES), lambda i: (0, 0)),
        out_shape=jax.ShapeDtypeStruct((8, _LANES), jnp.float32),
        scratch_shapes=[pltpu.VMEM((8, _LANES), jnp.float32),
                        pltpu.VMEM((_O, _ROWS, _LANES), jnp.float32),
                        pltpu.VMEM((2, _IMGS, _ROWS, _LANES), jnp.float32),
                        pltpu.VMEM((2, 2 * _IMGS, _LANES), jnp.float32)],
    )(targets, loc_p, conf_p, pri_p)
    return out[0, 0], out[1, 0]


# speedup vs baseline: 32.4460x; 1.0035x over previous
import jax
import jax.numpy as jnp
from jax.experimental import pallas as pl
from jax.experimental.pallas import tpu as pltpu

_NUM_CLASSES = 21
_B, _P, _O = 32, 8732, 8
_ROWS, _LANES = 70, 128
_PP = _ROWS * _LANES
_THRESHOLD = 0.5
_V0, _V1 = 0.1, 0.2
_IMGS = 4
_STEPS = _B // _IMGS


def _body(targets_ref, loc_ref, conf_ref, pri_ref, out_ref, acc_ref, ov_scr,
          lcr_scr, meta_scr):
    i = pl.program_id(0)

    @pl.when(i == 0)
    def _():
        acc_ref[...] = jnp.zeros((8, _LANES), jnp.float32)

    cur = jax.lax.rem(i, 2)
    prv = jax.lax.rem(i + 1, 2)

    ones = jnp.ones((_LANES, _LANES), jnp.float32)
    ones_bf = jnp.ones((_LANES, _LANES), jnp.bfloat16)

    def rowsums(rows):
        stacked = jnp.concatenate(rows, axis=0)
        return jax.lax.dot_general(stacked, ones, (((1,), (0,)), ((), ())),
                                   preferred_element_type=jnp.float32)

    def rowcounts(rows):
        stacked = jnp.concatenate(rows, axis=0).astype(jnp.bfloat16)
        return jax.lax.dot_general(stacked, ones_bf, (((1,), (0,)), ((), ())),
                                   preferred_element_type=jnp.float32)

    def part(x):
        return jnp.sum(x, axis=0, keepdims=True)

    meta = meta_scr[prv]
    ks = [jax.lax.slice(meta, (m, 0), (m + 1, _LANES)) for m in range(_IMGS)]
    splca = [jax.lax.slice(meta, (_IMGS + m, 0), (_IMGS + m + 1, _LANES))
             for m in range(_IMGS)]
    bits = [jax.lax.bitcast_convert_type(lcr_scr[prv, m], jnp.int32)
            for m in range(_IMGS)]
    ts = [jnp.zeros((1, _LANES), jnp.int32) for _ in range(_IMGS)]

    def partcount(m, cnd):
        mask = bits[m] >= jnp.broadcast_to(cnd, (_ROWS, _LANES))
        return part(mask.astype(jnp.float32))

    def do_level(lvl):
        if lvl == 0:
            c30 = [t + (1 << 30) for t in ts]
            cnts = rowcounts([partcount(m, c30[m]) for m in range(_IMGS)])
            for m in range(_IMGS):
                ok = jax.lax.slice(cnts, (m, 0), (m + 1, _LANES)) >= ks[m]
                ts[m] = jnp.where(ok, c30[m], ts[m])
            return
        lo = 28 - 2 * (lvl - 1)
        cands = [[t + (d << lo) for d in (1, 2, 3)] for t in ts]
        parts = [partcount(m, cands[m][d]) for m in range(_IMGS)
                 for d in range(3)]
        cnts = rowcounts(parts)
        for m in range(_IMGS):
            ok1 = jax.lax.slice(cnts, (3 * m, 0), (3 * m + 1, _LANES)) >= ks[m]
            ok2 = jax.lax.slice(cnts, (3 * m + 1, 0), (3 * m + 2, _LANES)) >= ks[m]
            ok3 = jax.lax.slice(cnts, (3 * m + 2, 0), (3 * m + 3, _LANES)) >= ks[m]
            ts[m] = jnp.where(ok3, cands[m][2],
                              jnp.where(ok2, cands[m][1],
                                        jnp.where(ok1, cands[m][0], ts[m])))

    lvl_counter = [0]

    def step_level():
        if lvl_counter[0] < 16:
            do_level(lvl_counter[0])
            lvl_counter[0] += 1

    lin = (jax.lax.broadcasted_iota(jnp.int32, (_ROWS, _LANES), 0) * _LANES
           + jax.lax.broadcasted_iota(jnp.int32, (_ROWS, _LANES), 1)
           ).astype(jnp.float32)
    valid = lin < float(_P)

    pcx = pri_ref[0]
    pcy = pri_ref[1]
    pw = pri_ref[2]
    ph = pri_ref[3]
    px1 = pcx - pw * 0.5
    py1 = pcy - ph * 0.5
    px2 = pcx + pw * 0.5
    py2 = pcy + ph * 0.5
    area_p = (px2 - px1) * (py2 - py1)

    ra_parts = []
    for m in range(_IMGS):
        bt_ov = jnp.full((_ROWS, _LANES), -1.0, jnp.float32)
        bt_idx = jnp.zeros((_ROWS, _LANES), jnp.float32)
        tco = []
        pmax = []
        for j in range(_O):
            tx1 = targets_ref[m, j, 0]
            ty1 = targets_ref[m, j, 1]
            tx2 = targets_ref[m, j, 2]
            ty2 = targets_ref[m, j, 3]
            lab = targets_ref[m, j, 4]
            tco.append((tx1, ty1, tx2, ty2, lab))
            iw = jnp.maximum(jnp.minimum(px2, tx2) - jnp.maximum(px1, tx1), 0.0)
            ih = jnp.maximum(jnp.minimum(py2, ty2) - jnp.maximum(py1, ty1), 0.0)
            inter = iw * ih
            area_t = (tx2 - tx1) * (ty2 - ty1)
            ov = inter / (area_t + area_p - inter)
            ov = jnp.where(valid, ov, -1.0)
            ov_scr[j] = ov
            upd = ov > bt_ov
            bt_ov = jnp.where(upd, ov, bt_ov)
            bt_idx = jnp.where(upd, float(j), bt_idx)
            pmax.append(jnp.max(ov, axis=0, keepdims=True))

        step_level()

        m8 = jnp.max(jnp.concatenate(pmax, axis=0), axis=1, keepdims=True)
        m8b = jnp.broadcast_to(m8, (_O, _LANES))
        pidx = []
        for j in range(_O):
            mj = jnp.broadcast_to(
                jax.lax.slice(m8b, (j, 0), (j + 1, _LANES)), (_ROWS, _LANES))
            cand = jnp.where(ov_scr[j] == mj, lin, float(_PP))
            pidx.append(jnp.min(cand, axis=0, keepdims=True))
        bpi8 = jnp.min(jnp.concatenate(pidx, axis=0), axis=1, keepdims=True)
        bpi8b = jnp.broadcast_to(bpi8, (_O, _LANES))

        for j in range(_O):
            bj = jnp.broadcast_to(
                jax.lax.slice(bpi8b, (j, 0), (j + 1, _LANES)), (_ROWS, _LANES))
            f = lin == bj
            bt_ov = jnp.where(f, 2.0, bt_ov)
            bt_idx = jnp.where(f, float(j), bt_idx)

        step_level()

        mx1 = jnp.zeros_like(lin)
        my1 = jnp.zeros_like(lin)
        mx2 = jnp.zeros_like(lin)
        my2 = jnp.zeros_like(lin)
        mlab = jnp.zeros_like(lin)
        for j in range(_O):
            sel = bt_idx == float(j)
            tx1, ty1, tx2, ty2, lab = tco[j]
            mx1 = jnp.where(sel, tx1, mx1)
            my1 = jnp.where(sel, ty1, my1)
            mx2 = jnp.where(sel, tx2, mx2)
            my2 = jnp.where(sel, ty2, my2)
            mlab = jnp.where(sel, lab, mlab)

        cf = jnp.where(bt_ov < _THRESHOLD, 0.0, mlab + 1.0)
        pos = cf > 0.0
        posf = pos.astype(jnp.float32)

        g_cx = ((mx1 + mx2) * 0.5 - pcx) / (_V0 * pw)
        g_cy = ((my1 + my2) * 0.5 - pcy) / (_V0 * ph)
        g_w = jnp.log((mx2 - mx1) / pw) / _V1
        g_h = jnp.log((my2 - my1) / ph) / _V1
        sl = jnp.zeros_like(lin)
        for c, g in enumerate((g_cx, g_cy, g_w, g_h)):
            d = jnp.abs(loc_ref[m, c] - g)
            sl = sl + jnp.where(d < 1.0, 0.5 * d * d, d - 0.5)

        step_level()

        cmax = conf_ref[m, 0]
        for c in range(1, _NUM_CLASSES):
            cmax = jnp.maximum(cmax, conf_ref[m, c])
        s = jnp.zeros_like(lin)
        g = jnp.zeros_like(lin)
        for c in range(_NUM_CLASSES):
            row = conf_ref[m, c]
            s = s + jnp.exp(row - cmax)
            g = jnp.where(cf == float(c), row, g)
        lse = jnp.log(s) + cmax
        lca = jnp.where(valid, lse - g, 0.0)
        lcr_scr[cur, m] = jnp.where(pos, 0.0, lca)

        ra_parts += [part(posf), part(lca * posf), part(sl * posf)]

        step_level()

    while lvl_counter[0] < 16:
        step_level()

    rb_parts = []
    tvals = []
    for m in range(_IMGS):
        tval = jax.lax.bitcast_convert_type(ts[m], jnp.float32)
        tvals.append(tval)
        lcr = lcr_scr[prv, m]
        gt = lcr > jnp.broadcast_to(tval, (_ROWS, _LANES))
        gtf = gt.astype(jnp.float32)
        rb_parts += [part(gtf), part(gtf * lcr)]
    rb = rowsums(rb_parts)

    lossc_t = jnp.zeros((1, _LANES), jnp.float32)
    for m in range(_IMGS):
        m_cnt = jax.lax.slice(rb, (2 * m, 0), (2 * m + 1, _LANES))
        sum_gt = jax.lax.slice(rb, (2 * m + 1, 0), (2 * m + 2, _LANES))
        lossc_t = lossc_t + splca[m] + sum_gt + (ks[m] - m_cnt) * tvals[m]
    lossc_t = jnp.where(i > 0, lossc_t, jnp.zeros_like(lossc_t))

    ra = rowsums(ra_parts)
    lossl_t = jnp.zeros((1, _LANES), jnp.float32)
    npos_t = jnp.zeros((1, _LANES), jnp.float32)
    meta_rows = []
    for m in range(_IMGS):
        num_pos = jax.lax.slice(ra, (3 * m, 0), (3 * m + 1, _LANES))
        npos_t = npos_t + num_pos
        lossl_t = lossl_t + jax.lax.slice(ra, (3 * m + 2, 0), (3 * m + 3, _LANES))
        meta_rows.append(jnp.minimum(3.0 * num_pos, float(_P - 1)))
    for m in range(_IMGS):
        meta_rows.append(jax.lax.slice(ra, (3 * m + 1, 0), (3 * m + 2, _LANES)))
    meta_scr[cur] = jnp.concatenate(meta_rows, axis=0)

    zrow = jnp.zeros((1, _LANES), jnp.float32)
    contrib = jnp.concatenate(
        [jnp.where(i < _STEPS, lossl_t, zrow), lossc_t,
         jnp.where(i < _STEPS, npos_t, zrow),
         jnp.zeros((5, _LANES), jnp.float32)], axis=0)
    acc = acc_ref[...] + contrib
    acc_ref[...] = acc

    @pl.when(i == _STEPS)
    def _():
        n = jnp.broadcast_to(jax.lax.slice(acc, (2, 0), (3, _LANES)),
                             (8, _LANES))
        out_ref[...] = acc / n


def kernel(loc_data, conf_data, priors, targets):
    pad = _PP - _P
    loc_p = jnp.concatenate(
        [jnp.swapaxes(loc_data, 1, 2),
         jnp.zeros((_B, 4, pad), jnp.float32)],
        axis=2).reshape(_B, 4, _ROWS, _LANES)
    conf_p = jnp.concatenate(
        [jnp.swapaxes(conf_data, 1, 2),
         jnp.zeros((_B, _NUM_CLASSES, pad), jnp.float32)],
        axis=2).reshape(_B, _NUM_CLASSES, _ROWS, _LANES)
    pri_p = jnp.pad(priors.T, ((0, 0), (0, pad)),
                    constant_values=0.5).reshape(4, _ROWS, _LANES)

    def clamp(i):
        return jnp.minimum(i, _STEPS - 1)

    out = pl.pallas_call(
        _body,
        grid=(_STEPS + 1,),
        in_specs=[
            pl.BlockSpec((_IMGS, _O, 5), lambda i: (clamp(i), 0, 0),
                         memory_space=pltpu.SMEM),
            pl.BlockSpec((_IMGS, 4, _ROWS, _LANES),
                         lambda i: (clamp(i), 0, 0, 0)),
            pl.BlockSpec((_IMGS, _NUM_CLASSES, _ROWS, _LANES),
                         lambda i: (clamp(i), 0, 0, 0)),
            pl.BlockSpec((4, _ROWS, _LANES), lambda i: (0, 0, 0)),
        ],
        out_specs=pl.BlockSpec((8, _LANES), lambda i: (0, 0)),
        out_shape=jax.ShapeDtypeStruct((8, _LANES), jnp.float32),
        scratch_shapes=[pltpu.VMEM((8, _LANES), jnp.float32),
                        pltpu.VMEM((_O, _ROWS, _LANES), jnp.float32),
                        pltpu.VMEM((2, _IMGS, _ROWS, _LANES), jnp.float32),
                        pltpu.VMEM((2, 2 * _IMGS, _LANES), jnp.float32)],
    )(targets, loc_p, conf_p, pri_p)
    return out[0, 0], out[1, 0]
